# baseline (speedup 1.0000x reference)
import functools

import jax
import jax.numpy as jnp
from jax import lax
from jax.experimental import pallas as pl
from jax.experimental.pallas import tpu as pltpu
from jax.experimental.pallas import tpu_sc as plsc

_TC_BLOCK_ROWS = 25000
_SC_CHUNK = 256


def _tc_copy(embed_user):
    n, e = embed_user.shape
    grid = (-(-n // _TC_BLOCK_ROWS),)
    spec = pl.BlockSpec((_TC_BLOCK_ROWS, e), lambda i: (i, 0))
    return pl.pallas_call(
        lambda u_ref, o_ref: o_ref.__setitem__(..., u_ref[...]),
        grid=grid,
        in_specs=[spec],
        out_specs=spec,
        out_shape=jax.ShapeDtypeStruct((n, e), embed_user.dtype),
    )(embed_user)


def _sc_copy(embed_item):
    n, e = embed_item.shape
    info = plsc.get_sparse_core_info()
    nw = info.num_cores * info.num_subcores
    rows_main = -(-n // nw)
    rows_main += (-rows_main) % 8
    rows_last = n - (nw - 1) * rows_main
    assert rows_last > 0
    mesh = plsc.VectorSubcoreMesh(core_axis_name="c", subcore_axis_name="s")

    def chunk_list(total):
        out, off = [], 0
        while off < total:
            out.append((off, min(_SC_CHUNK, total - off)))
            off += _SC_CHUNK
        return out

    @functools.partial(
        pl.kernel,
        mesh=mesh,
        out_type=jax.ShapeDtypeStruct((n, e), embed_item.dtype),
        scratch_types=[
            pltpu.VMEM((2, _SC_CHUNK, 128), jnp.float32),
            pltpu.SemaphoreType.DMA((2,)),
            pltpu.SemaphoreType.DMA((2,)),
        ],
    )
    def sc_copy(i_hbm, oi_hbm, buf, sem_in, sem_out):
        wid = lax.axis_index("s") * info.num_cores + lax.axis_index("c")
        base = pl.multiple_of(wid * rows_main, 8)

        def pipelined_copy(rows):
            chunks = chunk_list(rows)
            n_c = len(chunks)

            def fill(c):
                off, sz = chunks[c]
                return pltpu.make_async_copy(
                    i_hbm.at[pl.ds(base + off, sz)],
                    buf.at[c % 2, pl.ds(0, sz)], sem_in.at[c % 2])

            def drain(c):
                off, sz = chunks[c]
                return pltpu.make_async_copy(
                    buf.at[c % 2, pl.ds(0, sz)],
                    oi_hbm.at[pl.ds(base + off, sz)], sem_out.at[c % 2])

            fill(0).start()
            for c in range(n_c):
                fill(c).wait()
                if c + 1 < n_c:
                    if c >= 1:
                        drain(c - 1).wait()
                    fill(c + 1).start()
                drain(c).start()
            for c in range(max(0, n_c - 2), n_c):
                drain(c).wait()

        @pl.when(wid < nw - 1)
        def _main():
            pipelined_copy(rows_main)

        @pl.when(wid == nw - 1)
        def _tail():
            pipelined_copy(rows_last)

    return sc_copy(embed_item)


def kernel(embed_user, embed_item):
    out_item = _sc_copy(embed_item)
    out_user = _tc_copy(embed_user)
    return (out_user, out_item)


# speedup vs baseline: 1.3301x; 1.3301x over previous
import jax
import jax.numpy as jnp
from jax.experimental import pallas as pl

_BLOCK_ROWS = 15000


def _copy_body(u_ref, i_ref, ou_ref, oi_ref):
    ou_ref[...] = u_ref[...]
    oi_ref[...] = i_ref[...]


def kernel(embed_user, embed_item):
    n_u, e = embed_user.shape
    n_i, _ = embed_item.shape
    assert n_u == n_i, "single-grid copy assumes equal table heights"
    grid = (-(-n_u // _BLOCK_ROWS),)
    spec = pl.BlockSpec((_BLOCK_ROWS, e), lambda i: (i, 0))
    out_u, out_i = pl.pallas_call(
        _copy_body,
        grid=grid,
        in_specs=[spec, spec],
        out_specs=[spec, spec],
        out_shape=[
            jax.ShapeDtypeStruct((n_u, e), embed_user.dtype),
            jax.ShapeDtypeStruct((n_i, e), embed_item.dtype),
        ],
    )(embed_user, embed_item)
    return (out_u, out_i)
